# baseline (speedup 1.0000x reference)
import functools

import jax
import jax.numpy as jnp
from jax import lax
from jax.experimental import pallas as pl
from jax.experimental.pallas import tpu as pltpu
from jax.experimental.pallas import tpu_sc as plsc

VOCAB = 1000000
D = 64
DP = 128
S = 512
B = 1024
N = B * S

NC = 2
NS = 16
NW = NC * NS
PER_W = N // NW
CHUNK = 256
N_CHUNKS = PER_W // CHUNK
POS_PER_CHUNK = S // CHUNK


@functools.partial(
    pl.kernel,
    mesh=plsc.VectorSubcoreMesh(core_axis_name="c", subcore_axis_name="s"),
    out_type=jax.ShapeDtypeStruct((N // 2, DP), jnp.float32),
    compiler_params=pltpu.CompilerParams(needs_layout_passes=False),
    scratch_types=[
        pltpu.VMEM((CHUNK,), jnp.int32),
        pltpu.VMEM((CHUNK,), jnp.int32),
        pltpu.VMEM((CHUNK,), jnp.int32),
        pltpu.VMEM((CHUNK,), jnp.int32),
        pltpu.VMEM((CHUNK,), jnp.int32),
        pltpu.VMEM((CHUNK,), jnp.int32),
        pltpu.VMEM((CHUNK, DP), jnp.float32),
        pltpu.VMEM((CHUNK, DP), jnp.float32),
        pltpu.VMEM((S // 2, DP), jnp.float32),
        pltpu.VMEM((CHUNK // 2, DP), jnp.float32),
        pltpu.SemaphoreType.DMA,
        pltpu.SemaphoreType.DMA,
    ],
)
def _sc_embed(x_hbm, tok_hbm, pos_hbm, out_hbm, hidx0, hidx1, par0, par1,
              xb0, xb1, rows0, rows1, pos_v, comp_v, g0, g1):
    wid = lax.axis_index("s") * NC + lax.axis_index("c")
    base = wid * PER_W
    hidx_v = (hidx0, hidx1)
    par_v = (par0, par1)
    xb_v = (xb0, xb1)
    rows_v = (rows0, rows1)
    gsem = (g0, g1)

    iota = lax.iota(jnp.int32, 16)
    iotah = lax.shift_right_logical(iota, 1)
    parity64 = (iota & 1) * 64

    pltpu.sync_copy(pos_hbm, pos_v)

    def fetch(g, b):
        off = pl.multiple_of(base + g * CHUNK, CHUNK)
        pltpu.sync_copy(x_hbm.at[pl.ds(off, CHUNK)], xb_v[b])
        for l in range(CHUNK // 16):
            k = pl.ds(l * 16, 16)
            xv = xb_v[b][k]
            hidx_v[b][k] = lax.shift_right_logical(xv, 1)
            par_v[b][k] = (xv & 1) * 64
        for h in range(CHUNK // 128):
            pltpu.async_copy(
                tok_hbm.at[hidx_v[b].at[pl.ds(h * 128, 128)]],
                rows_v[b].at[pl.ds(h * 128, 128)], gsem[b])

    def drain(b):
        for h in range(CHUNK // 128):
            pltpu.make_async_copy(
                tok_hbm.at[hidx_v[b].at[pl.ds(h * 128, 128)]],
                rows_v[b].at[pl.ds(h * 128, 128)], gsem[b]).wait()

    fetch(0, 0)
    fetch(1, 1)

    def outer(c, _):
        for b in range(2):
            g = c * 2 + b
            rows = rows_v[b]
            drain(b)

            posh0 = lax.rem(g, POS_PER_CHUNK) * (CHUNK // 2)

            def group_body(t, _):
                rowt = t * 16 + iota
                ph = posh0 + t * 8 + iotah
                ch = t * 8 + iotah
                parv = par_v[b][pl.ds(t * 16, 16)]

                def d_body(dpos, _):
                    cg = parv + dpos
                    cl = parity64 + dpos
                    tokv = plsc.load_gather(rows, [rowt, cg])
                    posv = plsc.load_gather(pos_v, [ph, cl])
                    plsc.store_scatter(comp_v, [ch, cl], tokv + posv)
                    return ()

                lax.fori_loop(0, D, d_body, (), unroll=4)
                return ()

            lax.fori_loop(0, CHUNK // 16, group_body, ())

            off2 = pl.multiple_of((base + g * CHUNK) // 2, CHUNK // 2)
            pltpu.sync_copy(comp_v, out_hbm.at[pl.ds(off2, CHUNK // 2)])

            @pl.when(g + 2 < N_CHUNKS)
            def _():
                fetch(g + 2, b)
        return ()

    lax.fori_loop(0, N_CHUNKS // 2, outer, ())


def kernel(x, token_table, pos_table):
    xf = x.reshape(-1).astype(jnp.int32)
    tok2 = token_table.reshape(VOCAB // 2, DP)
    pos2 = pos_table.reshape(S // 2, DP)
    out2 = _sc_embed(xf, tok2, pos2)
    return out2.reshape(B, S, D)


# speedup vs baseline: 1.2896x; 1.2896x over previous
import functools

import jax
import jax.numpy as jnp
from jax import lax
from jax.experimental import pallas as pl
from jax.experimental.pallas import tpu as pltpu
from jax.experimental.pallas import tpu_sc as plsc

VOCAB = 1000000
D = 64
DP = 128
S = 512
B = 1024
N = B * S

NC = 2
NS = 16
NW = NC * NS
PER_W = N // NW

N_STRIPES = (VOCAB - D) // DP
TAIL = VOCAB - N_STRIPES * DP
MAX_STRIPES_PER_W = (N_STRIPES + 2 * NW - 1) // (2 * NW)

CHUNK = 256
N_CHUNKS = PER_W // CHUNK
POS_PER_CHUNK = S // CHUNK


@functools.partial(
    pl.kernel,
    mesh=plsc.VectorSubcoreMesh(core_axis_name="c", subcore_axis_name="s"),
    out_type=jax.ShapeDtypeStruct((VOCAB, DP), jnp.float32),
    compiler_params=pltpu.CompilerParams(needs_layout_passes=False),
    scratch_types=[
        pltpu.VMEM((D, DP), jnp.float32),
        pltpu.VMEM((D, DP), jnp.float32),
        pltpu.VMEM((DP, DP), jnp.float32),
        pltpu.VMEM((DP, DP), jnp.float32),
        pltpu.VMEM((TAIL // 2, DP), jnp.float32),
        pltpu.SemaphoreType.DMA,
        pltpu.SemaphoreType.DMA,
        pltpu.SemaphoreType.DMA,
        pltpu.SemaphoreType.DMA,
    ],
)
def _sc_relayout(tokT_hbm, tail_hbm, scr_hbm, in0, in1, tr0, tr1, tail_v,
                 gi0, gi1, so0, so1):
    wid = lax.axis_index("s") * NC + lax.axis_index("c")
    in_v = (in0, in1)
    tr_v = (tr0, tr1)
    gsem = (gi0, gi1)
    ssem = (so0, so1)

    iota = lax.iota(jnp.int32, 16)
    dvecs = [c * 16 + iota for c in range(D // 16)]

    def stripe_of(i, b):
        return (2 * i + b) * NW + wid

    @pl.when(stripe_of(0, 0) < N_STRIPES)
    def _():
        vt = stripe_of(0, 0)
        pltpu.async_copy(tokT_hbm.at[:, pl.ds(vt * DP, DP)], in0, gi0)

    @pl.when(stripe_of(0, 1) < N_STRIPES)
    def _():
        vt = stripe_of(0, 1)
        pltpu.async_copy(tokT_hbm.at[:, pl.ds(vt * DP, DP)], in1, gi1)

    def outer(i, _):
        for b in range(2):
            vt = stripe_of(i, b)

            @pl.when(vt < N_STRIPES)
            def _():
                inv = in_v[b]
                trv = tr_v[b]
                pltpu.make_async_copy(
                    tokT_hbm.at[:, pl.ds(vt * DP, DP)], inv, gsem[b]).wait()

                def col_body(vloc, _):
                    vs = jnp.full((16,), vloc, dtype=jnp.int32)
                    for c in range(D // 16):
                        trv[vloc, pl.ds(c * 16, 16)] = plsc.load_gather(
                            inv, [dvecs[c], vs])
                    return ()

                lax.fori_loop(0, DP, col_body, (), unroll=8)

                @pl.when(i > 0)
                def _():
                    pvt = stripe_of(i - 1, b)
                    pltpu.make_async_copy(
                        trv, scr_hbm.at[pl.ds(pvt * DP, DP)], ssem[b]).wait()

                pltpu.async_copy(trv, scr_hbm.at[pl.ds(vt * DP, DP)], ssem[b])

                nvt = stripe_of(i + 1, b)

                @pl.when(nvt < N_STRIPES)
                def _():
                    pltpu.async_copy(
                        tokT_hbm.at[:, pl.ds(nvt * DP, DP)], inv, gsem[b])
        return ()

    lax.fori_loop(0, MAX_STRIPES_PER_W, outer, ())

    for b in range(2):
        n_done = (N_STRIPES - wid - b * NW + 2 * NW - 1) // (2 * NW)

        @pl.when(n_done > 0)
        def _():
            lvt = (2 * (n_done - 1) + b) * NW + wid
            pltpu.make_async_copy(
                tr_v[b], scr_hbm.at[pl.ds(lvt * DP, DP)], ssem[b]).wait()

    @pl.when(wid == 0)
    def _():
        pltpu.sync_copy(tail_hbm, tail_v)
        for v in range(TAIL):
            half = (v % 2) * 64
            for c in range(D // 16):
                tr0[v % DP, pl.ds(c * 16, 16)] = \
                    tail_v[v // 2, pl.ds(half + c * 16, 16)]
        pltpu.sync_copy(tr0.at[pl.ds(0, TAIL)],
                        scr_hbm.at[pl.ds(N_STRIPES * DP, TAIL)])


@functools.partial(
    pl.kernel,
    mesh=plsc.VectorSubcoreMesh(core_axis_name="c", subcore_axis_name="s"),
    out_type=jax.ShapeDtypeStruct((N // 2, DP), jnp.float32),
    compiler_params=pltpu.CompilerParams(needs_layout_passes=False),
    scratch_types=[
        pltpu.VMEM((CHUNK,), jnp.int32),
        pltpu.VMEM((CHUNK,), jnp.int32),
        pltpu.VMEM((CHUNK, DP), jnp.float32),
        pltpu.VMEM((CHUNK, DP), jnp.float32),
        pltpu.VMEM((S // 2, DP), jnp.float32),
        pltpu.VMEM((CHUNK // 2, DP), jnp.float32),
        pltpu.SemaphoreType.DMA,
        pltpu.SemaphoreType.DMA,
    ],
)
def _sc_gather(x_hbm, scr_hbm, pos_hbm, out_hbm, ix0, ix1, rows0, rows1,
               pos_v, comp_v, g0, g1):
    wid = lax.axis_index("s") * NC + lax.axis_index("c")
    base = wid * PER_W
    ix_v = (ix0, ix1)
    rows_v = (rows0, rows1)
    gsem = (g0, g1)

    pltpu.sync_copy(pos_hbm, pos_v)

    def fetch(g, b):
        off = pl.multiple_of(base + g * CHUNK, CHUNK)
        pltpu.sync_copy(x_hbm.at[pl.ds(off, CHUNK)], ix_v[b])
        for h in range(CHUNK // 128):
            pltpu.async_copy(
                scr_hbm.at[ix_v[b].at[pl.ds(h * 128, 128)]],
                rows_v[b].at[pl.ds(h * 128, 128)], gsem[b])

    def drain(b):
        for h in range(CHUNK // 128):
            pltpu.make_async_copy(
                scr_hbm.at[ix_v[b].at[pl.ds(h * 128, 128)]],
                rows_v[b].at[pl.ds(h * 128, 128)], gsem[b]).wait()

    fetch(0, 0)
    fetch(1, 1)

    def outer(c, _):
        for b in range(2):
            g = c * 2 + b
            rows = rows_v[b]
            drain(b)

            ph = lax.rem(g, POS_PER_CHUNK) * (CHUNK // 2)

            def add_body(r, _):
                rh = lax.shift_right_logical(r, 1)
                half = (r & 1) * 64
                for d in range(D // 16):
                    src = pl.ds(d * 16, 16)
                    dst = pl.ds(half + d * 16, 16)
                    comp_v[rh, dst] = rows[r, src] + pos_v[ph + rh, dst]
                return ()

            lax.fori_loop(0, CHUNK, add_body, (), unroll=4)

            off2 = pl.multiple_of((base + g * CHUNK) // 2, CHUNK // 2)
            pltpu.sync_copy(comp_v, out_hbm.at[pl.ds(off2, CHUNK // 2)])

            @pl.when(g + 2 < N_CHUNKS)
            def _():
                fetch(g + 2, b)
        return ()

    lax.fori_loop(0, N_CHUNKS // 2, outer, ())


def kernel(x, token_table, pos_table):
    xf = x.reshape(-1).astype(jnp.int32)
    tokT = token_table.T
    tail2 = token_table[VOCAB - TAIL:].reshape(TAIL // 2, DP)
    scr = _sc_relayout(tokT, tail2)
    pos2 = pos_table.reshape(S // 2, DP)
    out2 = _sc_gather(xf, scr, pos2)
    return out2.reshape(B, S, D)


# speedup vs baseline: 2.2393x; 1.7364x over previous
import functools

import jax
import jax.numpy as jnp
from jax import lax
from jax.experimental import pallas as pl
from jax.experimental.pallas import tpu as pltpu
from jax.experimental.pallas import tpu_sc as plsc

VOCAB = 1000000
D = 64
DP = 128
S = 512
B = 1024
N = B * S

NC = 2
NS = 16
NW = NC * NS
PER_W = N // NW

N_STRIPES = (VOCAB - D) // DP
TAIL = VOCAB - N_STRIPES * DP
MAX_STRIPES_PER_W = (N_STRIPES + 2 * NW - 1) // (2 * NW)

CHUNK = 256
N_CHUNKS = PER_W // CHUNK
POS_PER_CHUNK = S // CHUNK


@functools.partial(
    pl.kernel,
    mesh=plsc.VectorSubcoreMesh(core_axis_name="c", subcore_axis_name="s"),
    out_type=jax.ShapeDtypeStruct((VOCAB, DP), jnp.float32),
    compiler_params=pltpu.CompilerParams(needs_layout_passes=False),
    scratch_types=[
        pltpu.VMEM((D, DP), jnp.float32),
        pltpu.VMEM((D, DP), jnp.float32),
        pltpu.VMEM((DP, DP), jnp.float32),
        pltpu.VMEM((DP, DP), jnp.float32),
        pltpu.VMEM((TAIL // 2, DP), jnp.float32),
        pltpu.SemaphoreType.DMA,
        pltpu.SemaphoreType.DMA,
        pltpu.SemaphoreType.DMA,
        pltpu.SemaphoreType.DMA,
    ],
)
def _sc_relayout(tokT_hbm, tail_hbm, scr_hbm, in0, in1, tr0, tr1, tail_v,
                 gi0, gi1, so0, so1):
    wid = lax.axis_index("s") * NC + lax.axis_index("c")
    in_v = (in0, in1)
    tr_v = (tr0, tr1)
    gsem = (gi0, gi1)
    ssem = (so0, so1)

    iota = lax.iota(jnp.int32, 16)
    dvecs = [c * 16 + iota for c in range(D // 16)]

    def stripe_of(i, b):
        return (2 * i + b) * NW + wid

    @pl.when(stripe_of(0, 0) < N_STRIPES)
    def _():
        vt = stripe_of(0, 0)
        pltpu.async_copy(tokT_hbm.at[:, pl.ds(vt * DP, DP)], in0, gi0)

    @pl.when(stripe_of(0, 1) < N_STRIPES)
    def _():
        vt = stripe_of(0, 1)
        pltpu.async_copy(tokT_hbm.at[:, pl.ds(vt * DP, DP)], in1, gi1)

    def outer(i, _):
        for b in range(2):
            vt = stripe_of(i, b)

            @pl.when(vt < N_STRIPES)
            def _():
                inv = in_v[b]
                trv = tr_v[b]
                pltpu.make_async_copy(
                    tokT_hbm.at[:, pl.ds(vt * DP, DP)], inv, gsem[b]).wait()

                @plsc.parallel_loop(0, DP, step=1, unroll=8)
                def col_body(vloc):
                    vs = jnp.full((16,), vloc, dtype=jnp.int32)
                    for c in range(D // 16):
                        trv[vloc, pl.ds(c * 16, 16)] = plsc.load_gather(
                            inv, [dvecs[c], vs])

                @pl.when(i > 0)
                def _():
                    pvt = stripe_of(i - 1, b)
                    pltpu.make_async_copy(
                        trv, scr_hbm.at[pl.ds(pvt * DP, DP)], ssem[b]).wait()

                pltpu.async_copy(trv, scr_hbm.at[pl.ds(vt * DP, DP)], ssem[b])

                nvt = stripe_of(i + 1, b)

                @pl.when(nvt < N_STRIPES)
                def _():
                    pltpu.async_copy(
                        tokT_hbm.at[:, pl.ds(nvt * DP, DP)], inv, gsem[b])
        return ()

    lax.fori_loop(0, MAX_STRIPES_PER_W, outer, ())

    for b in range(2):
        n_done = (N_STRIPES - wid - b * NW + 2 * NW - 1) // (2 * NW)

        @pl.when(n_done > 0)
        def _():
            lvt = (2 * (n_done - 1) + b) * NW + wid
            pltpu.make_async_copy(
                tr_v[b], scr_hbm.at[pl.ds(lvt * DP, DP)], ssem[b]).wait()

    @pl.when(wid == 0)
    def _():
        pltpu.sync_copy(tail_hbm, tail_v)
        for v in range(TAIL):
            half = (v % 2) * 64
            for c in range(D // 16):
                tr0[v % DP, pl.ds(c * 16, 16)] = \
                    tail_v[v // 2, pl.ds(half + c * 16, 16)]
        pltpu.sync_copy(tr0.at[pl.ds(0, TAIL)],
                        scr_hbm.at[pl.ds(N_STRIPES * DP, TAIL)])


@functools.partial(
    pl.kernel,
    mesh=plsc.VectorSubcoreMesh(core_axis_name="c", subcore_axis_name="s"),
    out_type=jax.ShapeDtypeStruct((N // 2, DP), jnp.float32),
    compiler_params=pltpu.CompilerParams(needs_layout_passes=False),
    scratch_types=[
        pltpu.VMEM((CHUNK,), jnp.int32),
        pltpu.VMEM((CHUNK,), jnp.int32),
        pltpu.VMEM((CHUNK, DP), jnp.float32),
        pltpu.VMEM((CHUNK, DP), jnp.float32),
        pltpu.VMEM((S // 2, DP), jnp.float32),
        pltpu.VMEM((CHUNK // 2, DP), jnp.float32),
        pltpu.SemaphoreType.DMA,
        pltpu.SemaphoreType.DMA,
    ],
)
def _sc_gather(x_hbm, scr_hbm, pos_hbm, out_hbm, ix0, ix1, rows0, rows1,
               pos_v, comp_v, g0, g1):
    wid = lax.axis_index("s") * NC + lax.axis_index("c")
    base = wid * PER_W
    ix_v = (ix0, ix1)
    rows_v = (rows0, rows1)
    gsem = (g0, g1)

    pltpu.sync_copy(pos_hbm, pos_v)

    def fetch(g, b):
        off = pl.multiple_of(base + g * CHUNK, CHUNK)
        pltpu.sync_copy(x_hbm.at[pl.ds(off, CHUNK)], ix_v[b])
        for h in range(CHUNK // 128):
            pltpu.async_copy(
                scr_hbm.at[ix_v[b].at[pl.ds(h * 128, 128)]],
                rows_v[b].at[pl.ds(h * 128, 128)], gsem[b])

    def drain(b):
        for h in range(CHUNK // 128):
            pltpu.make_async_copy(
                scr_hbm.at[ix_v[b].at[pl.ds(h * 128, 128)]],
                rows_v[b].at[pl.ds(h * 128, 128)], gsem[b]).wait()

    fetch(0, 0)
    fetch(1, 1)

    def outer(c, _):
        for b in range(2):
            g = c * 2 + b
            rows = rows_v[b]
            drain(b)

            ph = lax.rem(g, POS_PER_CHUNK) * (CHUNK // 2)

            @plsc.parallel_loop(0, CHUNK, step=1, unroll=4)
            def add_body(r):
                rh = lax.shift_right_logical(r, 1)
                half = (r & 1) * 64
                for d in range(D // 16):
                    src = pl.ds(d * 16, 16)
                    dst = pl.ds(half + d * 16, 16)
                    comp_v[rh, dst] = rows[r, src] + pos_v[ph + rh, dst]

            off2 = pl.multiple_of((base + g * CHUNK) // 2, CHUNK // 2)
            pltpu.sync_copy(comp_v, out_hbm.at[pl.ds(off2, CHUNK // 2)])

            @pl.when(g + 2 < N_CHUNKS)
            def _():
                fetch(g + 2, b)
        return ()

    lax.fori_loop(0, N_CHUNKS // 2, outer, ())


def kernel(x, token_table, pos_table):
    xf = x.reshape(-1).astype(jnp.int32)
    tokT = token_table.T
    tail2 = token_table[VOCAB - TAIL:].reshape(TAIL // 2, DP)
    scr = _sc_relayout(tokT, tail2)
    pos2 = pos_table.reshape(S // 2, DP)
    out2 = _sc_gather(xf, scr, pos2)
    return out2.reshape(B, S, D)
